# Initial kernel scaffold; baseline (speedup 1.0000x reference)
import jax
import jax.numpy as jnp
from jax.experimental import pallas as pl


def kernel(x, edge_index, W1, b1, W2, b2):
    raise NotImplementedError("write your pallas kernel here")


# speedup vs baseline: 27.9047x; 27.9047x over previous
import jax
import jax.numpy as jnp
from jax import lax
from jax.experimental import pallas as pl
from jax.experimental.pallas import tpu as pltpu
from jax.experimental.pallas import tpu_sc as plsc

N = 10000
E = 320000
D = 128
H = D // 2

NC, NS, L = 2, 16, 16
NT = NC * NS
EPT = E // NT
EPS = E // NS
CH = 125
NCHUNK = EPS // CH
SUB = 624

_MESH = plsc.VectorSubcoreMesh(
    core_axis_name="c", subcore_axis_name="s", num_cores=NC, num_subcores=NS)


def _deg_body(col_hbm, degp_hbm, col_v, hist):
    c = lax.axis_index("c")
    s = lax.axis_index("s")
    t = c * NS + s
    pltpu.sync_copy(col_hbm.at[pl.ds(t * EPT, EPT)], col_v)

    def zero(i, carry):
        hist[pl.ds(i * L, L)] = jnp.zeros((L,), jnp.float32)
        return carry

    lax.fori_loop(0, N // L, zero, 0)

    ones = jnp.ones((L,), jnp.float32)

    def acc(i, carry):
        idx = col_v[pl.ds(i * L, L)]
        plsc.addupdate_scatter(hist, [idx], ones)
        return carry

    lax.fori_loop(0, EPT // L, acc, 0)
    pltpu.sync_copy(hist, degp_hbm.at[t])


_deg_kernel = pl.kernel(
    _deg_body,
    out_type=jax.ShapeDtypeStruct((NT, N), jnp.float32),
    mesh=_MESH,
    compiler_params=pltpu.CompilerParams(needs_layout_passes=False),
    scratch_types=[
        pltpu.VMEM((EPT,), jnp.int32),
        pltpu.VMEM((N,), jnp.float32),
    ],
)


def _edge_body(y_hbm, row_hbm, col_hbm, out_hbm, row_v, col_v, g0, g1, zsh,
               sem0, sem1):
    c = lax.axis_index("c")
    s = lax.axis_index("s")
    yc = y_hbm.at[c]
    pltpu.sync_copy(row_hbm.at[pl.ds(s * NCHUNK, NCHUNK)], row_v)
    pltpu.sync_copy(col_hbm.at[pl.ds(s * NCHUNK, NCHUNK)], col_v)

    pltpu.sync_copy(yc.at[pl.ds(s * SUB, SUB)], zsh.at[pl.ds(s * SUB, SUB)])

    @pl.when(s == 0)
    def _():
        pltpu.sync_copy(yc.at[pl.ds(NS * SUB, N - NS * SUB)],
                        zsh.at[pl.ds(NS * SUB, N - NS * SUB)])

    plsc.subcore_barrier()

    pltpu.async_copy(yc.at[row_v.at[0]], g0, sem0)
    pltpu.async_copy(yc.at[row_v.at[1]], g1, sem1)

    def pair(i, carry):
        j0 = 2 * i
        pltpu.make_async_copy(yc.at[row_v.at[j0]], g0, sem0).wait()
        pltpu.sync_copy(g0, zsh.at[col_v.at[j0]], add=True)

        @pl.when(j0 + 2 < NCHUNK)
        def _():
            pltpu.async_copy(yc.at[row_v.at[j0 + 2]], g0, sem0)

        j1 = j0 + 1
        pltpu.make_async_copy(yc.at[row_v.at[j1]], g1, sem1).wait()
        pltpu.sync_copy(g1, zsh.at[col_v.at[j1]], add=True)

        @pl.when(j1 + 2 < NCHUNK)
        def _():
            pltpu.async_copy(yc.at[row_v.at[j1 + 2]], g1, sem1)

        return carry

    lax.fori_loop(0, NCHUNK // 2, pair, 0)
    plsc.subcore_barrier()
    pltpu.sync_copy(zsh.at[pl.ds(s * SUB, SUB)],
                    out_hbm.at[c, pl.ds(s * SUB, SUB)])

    @pl.when(s == 0)
    def _():
        pltpu.sync_copy(zsh.at[pl.ds(NS * SUB, N - NS * SUB)],
                        out_hbm.at[c, pl.ds(NS * SUB, N - NS * SUB)])


_edge_kernel = pl.kernel(
    _edge_body,
    out_type=jax.ShapeDtypeStruct((NC, N, H), jnp.float32),
    mesh=_MESH,
    compiler_params=pltpu.CompilerParams(use_tc_tiling_on_sc=False),
    scratch_types=[
        pltpu.VMEM((NCHUNK, CH), jnp.int32),
        pltpu.VMEM((NCHUNK, CH), jnp.int32),
        pltpu.VMEM((CH, H), jnp.float32),
        pltpu.VMEM((CH, H), jnp.float32),
        pltpu.VMEM_SHARED((N, H), jnp.float32),
        pltpu.SemaphoreType.DMA,
        pltpu.SemaphoreType.DMA,
    ],
)


_RB = 1000


def _dis_body(degp_ref, dis_ref):
    deg = jnp.sum(degp_ref[...], axis=0) + 1.0
    dis_ref[...] = lax.rsqrt(deg)


def _dis_kernel(degp):
    return pl.pallas_call(
        _dis_body,
        out_shape=jax.ShapeDtypeStruct((N,), jnp.float32),
    )(degp)


def _mm_body(x_ref, w_ref, dis_ref, y_ref):
    xw = jnp.dot(x_ref[...], w_ref[...], preferred_element_type=jnp.float32)
    y = dis_ref[...] * xw
    y_ref[0] = y[:, :H]
    y_ref[1] = y[:, H:]


def _mm_kernel(x, w, dis2):
    return pl.pallas_call(
        _mm_body,
        grid=(N // _RB,),
        in_specs=[
            pl.BlockSpec((_RB, D), lambda i: (i, 0)),
            pl.BlockSpec((D, D), lambda i: (0, 0)),
            pl.BlockSpec((_RB, 1), lambda i: (i, 0)),
        ],
        out_specs=pl.BlockSpec((NC, _RB, H), lambda i: (0, i, 0)),
        out_shape=jax.ShapeDtypeStruct((NC, N, H), jnp.float32),
    )(x, w, dis2)


def _mid_body(zp_ref, dis_ref, b_ref, w_ref, y2_ref):
    z = jnp.concatenate([zp_ref[0], zp_ref[1]], axis=1)
    h = jnp.maximum(dis_ref[...] * z + b_ref[...], 0.0)
    hw = jnp.dot(h, w_ref[...], preferred_element_type=jnp.float32)
    y2 = dis_ref[...] * hw
    y2_ref[0] = y2[:, :H]
    y2_ref[1] = y2[:, H:]


def _mid_kernel(zp, dis2, b1, w2):
    return pl.pallas_call(
        _mid_body,
        grid=(N // _RB,),
        in_specs=[
            pl.BlockSpec((NC, _RB, H), lambda i: (0, i, 0)),
            pl.BlockSpec((_RB, 1), lambda i: (i, 0)),
            pl.BlockSpec((1, D), lambda i: (0, 0)),
            pl.BlockSpec((D, D), lambda i: (0, 0)),
        ],
        out_specs=pl.BlockSpec((NC, _RB, H), lambda i: (0, i, 0)),
        out_shape=jax.ShapeDtypeStruct((NC, N, H), jnp.float32),
    )(zp, dis2, b1, w2)


def _out_body(zp_ref, dis_ref, b_ref, o_ref):
    z = jnp.concatenate([zp_ref[0], zp_ref[1]], axis=1)
    o_ref[...] = dis_ref[...] * z + b_ref[...]


def _out_kernel(zp, dis2, b2):
    return pl.pallas_call(
        _out_body,
        grid=(N // _RB,),
        in_specs=[
            pl.BlockSpec((NC, _RB, H), lambda i: (0, i, 0)),
            pl.BlockSpec((_RB, 1), lambda i: (i, 0)),
            pl.BlockSpec((1, D), lambda i: (0, 0)),
        ],
        out_specs=pl.BlockSpec((_RB, D), lambda i: (i, 0)),
        out_shape=jax.ShapeDtypeStruct((N, D), jnp.float32),
    )(zp, dis2, b2)


def kernel(x, edge_index, W1, b1, W2, b2):
    ei = edge_index.astype(jnp.int32)
    row2d = ei[0].reshape(E // CH, CH)
    col2d = ei[1].reshape(E // CH, CH)

    degp = _deg_kernel(ei[1])
    dis2 = _dis_kernel(degp).reshape(N, 1)

    y1 = _mm_kernel(x, W1, dis2)
    zp1 = _edge_kernel(y1, row2d, col2d)
    y2 = _mid_kernel(zp1, dis2, b1.reshape(1, D), W2)
    zp2 = _edge_kernel(y2, row2d, col2d)
    return _out_kernel(zp2, dis2, b2.reshape(1, D))


# speedup vs baseline: 32.0016x; 1.1468x over previous
import jax
import jax.numpy as jnp
from jax import lax
from jax.experimental import pallas as pl
from jax.experimental.pallas import tpu as pltpu
from jax.experimental.pallas import tpu_sc as plsc

N = 10000
E = 320000
D = 128
H = D // 2

NC, NS, L = 2, 16, 16
NT = NC * NS
EPT = E // NT
EPS = E // NS
CH = 125
NCHUNK = EPS // CH
SUB = 624

_MESH = plsc.VectorSubcoreMesh(
    core_axis_name="c", subcore_axis_name="s", num_cores=NC, num_subcores=NS)


def _deg_body(col_hbm, degp_hbm, col_v, hist):
    c = lax.axis_index("c")
    s = lax.axis_index("s")
    t = c * NS + s
    pltpu.sync_copy(col_hbm.at[pl.ds(t * EPT, EPT)], col_v)

    def zero(i, carry):
        hist[pl.ds(i * L, L)] = jnp.zeros((L,), jnp.float32)
        return carry

    lax.fori_loop(0, N // L, zero, 0)

    ones = jnp.ones((L,), jnp.float32)

    def acc(i, carry):
        idx = col_v[pl.ds(i * L, L)]
        plsc.addupdate_scatter(hist, [idx], ones)
        return carry

    lax.fori_loop(0, EPT // L, acc, 0)
    pltpu.sync_copy(hist, degp_hbm.at[t])


_deg_kernel = pl.kernel(
    _deg_body,
    out_type=jax.ShapeDtypeStruct((NT, N), jnp.float32),
    mesh=_MESH,
    compiler_params=pltpu.CompilerParams(needs_layout_passes=False),
    scratch_types=[
        pltpu.VMEM((EPT,), jnp.int32),
        pltpu.VMEM((N,), jnp.float32),
    ],
)


NBUF = 8
PREF = 4


def _edge_body(y_hbm, row_hbm, col_hbm, out_hbm, row_v, cbufs, gbufs, zsh,
               gsems, csems, ssems):
    c = lax.axis_index("c")
    s = lax.axis_index("s")
    yc = y_hbm.at[c]
    pltpu.sync_copy(row_hbm.at[pl.ds(s * NCHUNK, NCHUNK)], row_v)

    pltpu.sync_copy(yc.at[pl.ds(s * SUB, SUB)], zsh.at[pl.ds(s * SUB, SUB)])

    @pl.when(s == 0)
    def _():
        pltpu.sync_copy(yc.at[pl.ds(NS * SUB, N - NS * SUB)],
                        zsh.at[pl.ds(NS * SUB, N - NS * SUB)])

    plsc.subcore_barrier()

    def issue(j, b):
        pltpu.async_copy(yc.at[row_v.at[j]], gbufs[b], gsems[b])
        pltpu.async_copy(col_hbm.at[s * NCHUNK + j], cbufs[b], csems[b])

    for j in range(PREF):
        issue(j, j)

    def step(i, carry):
        j0 = NBUF * i
        for b in range(NBUF):
            j = j0 + b
            pltpu.make_async_copy(yc.at[row_v.at[j]], gbufs[b],
                                  gsems[b]).wait()
            pltpu.make_async_copy(col_hbm.at[s * NCHUNK + j], cbufs[b],
                                  csems[b]).wait()
            pltpu.async_copy(gbufs[b], zsh.at[cbufs[b]], ssems[b], add=True)
            b2 = (b + PREF) % NBUF

            @pl.when(j + PREF < NCHUNK)
            def _():
                @pl.when(j >= NBUF - PREF)
                def _():
                    pltpu.make_async_copy(
                        gbufs[b2], zsh.at[cbufs[b2]], ssems[b2]).wait()

                issue(j + PREF, b2)

        return carry

    lax.fori_loop(0, NCHUNK // NBUF, step, 0)
    for b in range(NBUF):
        pltpu.make_async_copy(gbufs[b], zsh.at[cbufs[b]], ssems[b]).wait()
    plsc.subcore_barrier()
    pltpu.sync_copy(zsh.at[pl.ds(s * SUB, SUB)],
                    out_hbm.at[c, pl.ds(s * SUB, SUB)])

    @pl.when(s == 0)
    def _():
        pltpu.sync_copy(zsh.at[pl.ds(NS * SUB, N - NS * SUB)],
                        out_hbm.at[c, pl.ds(NS * SUB, N - NS * SUB)])


_edge_kernel = pl.kernel(
    _edge_body,
    out_type=jax.ShapeDtypeStruct((NC, N, H), jnp.float32),
    mesh=_MESH,
    compiler_params=pltpu.CompilerParams(use_tc_tiling_on_sc=False),
    scratch_types=[
        pltpu.VMEM((NCHUNK, CH), jnp.int32),
        tuple(pltpu.VMEM((CH,), jnp.int32) for _ in range(NBUF)),
        tuple(pltpu.VMEM((CH, H), jnp.float32) for _ in range(NBUF)),
        pltpu.VMEM_SHARED((N, H), jnp.float32),
        tuple(pltpu.SemaphoreType.DMA for _ in range(NBUF)),
        tuple(pltpu.SemaphoreType.DMA for _ in range(NBUF)),
        tuple(pltpu.SemaphoreType.DMA for _ in range(NBUF)),
    ],
)


_RB = 1000


def _dis_body(degp_ref, dis_ref):
    deg = jnp.sum(degp_ref[...], axis=0) + 1.0
    dis_ref[...] = lax.rsqrt(deg)


def _dis_kernel(degp):
    return pl.pallas_call(
        _dis_body,
        out_shape=jax.ShapeDtypeStruct((N,), jnp.float32),
    )(degp)


def _mm_body(x_ref, w_ref, dis_ref, y_ref):
    xw = jnp.dot(x_ref[...], w_ref[...], preferred_element_type=jnp.float32)
    y = dis_ref[...] * xw
    y_ref[0] = y[:, :H]
    y_ref[1] = y[:, H:]


def _mm_kernel(x, w, dis2):
    return pl.pallas_call(
        _mm_body,
        grid=(N // _RB,),
        in_specs=[
            pl.BlockSpec((_RB, D), lambda i: (i, 0)),
            pl.BlockSpec((D, D), lambda i: (0, 0)),
            pl.BlockSpec((_RB, 1), lambda i: (i, 0)),
        ],
        out_specs=pl.BlockSpec((NC, _RB, H), lambda i: (0, i, 0)),
        out_shape=jax.ShapeDtypeStruct((NC, N, H), jnp.float32),
    )(x, w, dis2)


def _mid_body(zp_ref, dis_ref, b_ref, w_ref, y2_ref):
    z = jnp.concatenate([zp_ref[0], zp_ref[1]], axis=1)
    h = jnp.maximum(dis_ref[...] * z + b_ref[...], 0.0)
    hw = jnp.dot(h, w_ref[...], preferred_element_type=jnp.float32)
    y2 = dis_ref[...] * hw
    y2_ref[0] = y2[:, :H]
    y2_ref[1] = y2[:, H:]


def _mid_kernel(zp, dis2, b1, w2):
    return pl.pallas_call(
        _mid_body,
        grid=(N // _RB,),
        in_specs=[
            pl.BlockSpec((NC, _RB, H), lambda i: (0, i, 0)),
            pl.BlockSpec((_RB, 1), lambda i: (i, 0)),
            pl.BlockSpec((1, D), lambda i: (0, 0)),
            pl.BlockSpec((D, D), lambda i: (0, 0)),
        ],
        out_specs=pl.BlockSpec((NC, _RB, H), lambda i: (0, i, 0)),
        out_shape=jax.ShapeDtypeStruct((NC, N, H), jnp.float32),
    )(zp, dis2, b1, w2)


def _out_body(zp_ref, dis_ref, b_ref, o_ref):
    z = jnp.concatenate([zp_ref[0], zp_ref[1]], axis=1)
    o_ref[...] = dis_ref[...] * z + b_ref[...]


def _out_kernel(zp, dis2, b2):
    return pl.pallas_call(
        _out_body,
        grid=(N // _RB,),
        in_specs=[
            pl.BlockSpec((NC, _RB, H), lambda i: (0, i, 0)),
            pl.BlockSpec((_RB, 1), lambda i: (i, 0)),
            pl.BlockSpec((1, D), lambda i: (0, 0)),
        ],
        out_specs=pl.BlockSpec((_RB, D), lambda i: (i, 0)),
        out_shape=jax.ShapeDtypeStruct((N, D), jnp.float32),
    )(zp, dis2, b2)


def kernel(x, edge_index, W1, b1, W2, b2):
    ei = edge_index.astype(jnp.int32)
    row2d = ei[0].reshape(E // CH, CH)
    col2d = ei[1].reshape(E // CH, CH)

    degp = _deg_kernel(ei[1])
    dis2 = _dis_kernel(degp).reshape(N, 1)

    y1 = _mm_kernel(x, W1, dis2)
    zp1 = _edge_kernel(y1, row2d, col2d)
    y2 = _mid_kernel(zp1, dis2, b1.reshape(1, D), W2)
    zp2 = _edge_kernel(y2, row2d, col2d)
    return _out_kernel(zp2, dis2, b2.reshape(1, D))


# speedup vs baseline: 32.7060x; 1.0220x over previous
import jax
import jax.numpy as jnp
from jax import lax
from jax.experimental import pallas as pl
from jax.experimental.pallas import tpu as pltpu
from jax.experimental.pallas import tpu_sc as plsc

N = 10000
E = 320000
D = 128
H = D // 2

NC, NS, L = 2, 16, 16
NT = NC * NS
EPT = E // NT
EPS = E // NS
CH = 125
NCHUNK = EPS // CH
SUB = 624

_MESH = plsc.VectorSubcoreMesh(
    core_axis_name="c", subcore_axis_name="s", num_cores=NC, num_subcores=NS)


def _rsqrt16(d):
    i = plsc.bitcast(d, jnp.int32)
    i = 0x5F3759DF - lax.shift_right_arithmetic(i, 1)
    y = plsc.bitcast(i, jnp.float32)
    for _ in range(3):
        y = y * (1.5 - 0.5 * d * y * y)
    return y


def _deg_body(col_hbm, dis_hbm, col_v, hist, tmp, acc, hists_sh):
    c = lax.axis_index("c")
    s = lax.axis_index("s")
    pltpu.sync_copy(col_hbm.at[pl.ds(s * EPS, EPS)], col_v)

    def zero(i, carry):
        hist[pl.ds(i * L, L)] = jnp.zeros((L,), jnp.float32)
        return carry

    lax.fori_loop(0, N // L, zero, 0)

    ones = jnp.ones((L,), jnp.float32)

    def bump(i, carry):
        idx = col_v[pl.ds(i * L, L)]
        plsc.addupdate_scatter(hist, [idx], ones)
        return carry

    lax.fori_loop(0, EPS // L, bump, 0)
    pltpu.sync_copy(hist, hists_sh.at[s])
    plsc.subcore_barrier()

    def reduce_rows(lo, npart, buf_lo):
        def zacc(i, carry):
            acc[pl.ds(buf_lo + i * L, L)] = jnp.zeros((L,), jnp.float32)
            return carry

        lax.fori_loop(0, npart // L, zacc, 0)
        for k in range(NS):
            pltpu.sync_copy(hists_sh.at[k, pl.ds(lo, npart)],
                            tmp.at[pl.ds(buf_lo, npart)])

            def addk(i, carry):
                sl = pl.ds(buf_lo + i * L, L)
                acc[sl] = acc[sl] + tmp[sl]
                return carry

            lax.fori_loop(0, npart // L, addk, 0)

        def finish(i, carry):
            sl = pl.ds(buf_lo + i * L, L)
            acc[sl] = _rsqrt16(acc[sl] + 1.0)
            return carry

        lax.fori_loop(0, npart // L, finish, 0)

    reduce_rows(s * SUB, SUB, 0)

    @pl.when(s == 0)
    def _():
        reduce_rows(NS * SUB, N - NS * SUB, SUB)

    @pl.when(c == 0)
    def _():
        pltpu.sync_copy(acc.at[pl.ds(0, SUB)], dis_hbm.at[pl.ds(s * SUB, SUB)])

        @pl.when(s == 0)
        def _():
            pltpu.sync_copy(acc.at[pl.ds(SUB, N - NS * SUB)],
                            dis_hbm.at[pl.ds(NS * SUB, N - NS * SUB)])


_deg_kernel = pl.kernel(
    _deg_body,
    out_type=jax.ShapeDtypeStruct((N,), jnp.float32),
    mesh=_MESH,
    compiler_params=pltpu.CompilerParams(needs_layout_passes=False,
                                         use_tc_tiling_on_sc=False),
    scratch_types=[
        pltpu.VMEM((EPS,), jnp.int32),
        pltpu.VMEM((N,), jnp.float32),
        pltpu.VMEM((SUB + 16,), jnp.float32),
        pltpu.VMEM((SUB + 16,), jnp.float32),
        pltpu.VMEM_SHARED((NS, N), jnp.float32),
    ],
)


NBUF = 8
PREF = 6


def _edge_body(y_hbm, row_hbm, col_hbm, out_hbm, row_v, cbufs, gbufs, zsh,
               gsems, csems, ssems):
    c = lax.axis_index("c")
    s = lax.axis_index("s")
    yc = y_hbm.at[c]
    pltpu.sync_copy(row_hbm.at[pl.ds(s * NCHUNK, NCHUNK)], row_v)

    pltpu.sync_copy(yc.at[pl.ds(s * SUB, SUB)], zsh.at[pl.ds(s * SUB, SUB)])

    @pl.when(s == 0)
    def _():
        pltpu.sync_copy(yc.at[pl.ds(NS * SUB, N - NS * SUB)],
                        zsh.at[pl.ds(NS * SUB, N - NS * SUB)])

    plsc.subcore_barrier()

    def issue(j, b):
        pltpu.async_copy(yc.at[row_v.at[j]], gbufs[b], gsems[b])
        pltpu.async_copy(col_hbm.at[s * NCHUNK + j], cbufs[b], csems[b])

    for j in range(PREF):
        issue(j, j)

    def step(i, carry):
        j0 = NBUF * i
        for b in range(NBUF):
            j = j0 + b
            pltpu.make_async_copy(yc.at[row_v.at[j]], gbufs[b],
                                  gsems[b]).wait()
            pltpu.make_async_copy(col_hbm.at[s * NCHUNK + j], cbufs[b],
                                  csems[b]).wait()
            pltpu.async_copy(gbufs[b], zsh.at[cbufs[b]], ssems[b], add=True)
            b2 = (b + PREF) % NBUF

            @pl.when(j + PREF < NCHUNK)
            def _():
                @pl.when(j >= NBUF - PREF)
                def _():
                    pltpu.make_async_copy(
                        gbufs[b2], zsh.at[cbufs[b2]], ssems[b2]).wait()

                issue(j + PREF, b2)

        return carry

    lax.fori_loop(0, NCHUNK // NBUF, step, 0)
    for b in range(NBUF):
        pltpu.make_async_copy(gbufs[b], zsh.at[cbufs[b]], ssems[b]).wait()
    plsc.subcore_barrier()
    pltpu.sync_copy(zsh.at[pl.ds(s * SUB, SUB)],
                    out_hbm.at[c, pl.ds(s * SUB, SUB)])

    @pl.when(s == 0)
    def _():
        pltpu.sync_copy(zsh.at[pl.ds(NS * SUB, N - NS * SUB)],
                        out_hbm.at[c, pl.ds(NS * SUB, N - NS * SUB)])


_edge_kernel = pl.kernel(
    _edge_body,
    out_type=jax.ShapeDtypeStruct((NC, N, H), jnp.float32),
    mesh=_MESH,
    compiler_params=pltpu.CompilerParams(use_tc_tiling_on_sc=False),
    scratch_types=[
        pltpu.VMEM((NCHUNK, CH), jnp.int32),
        tuple(pltpu.VMEM((CH,), jnp.int32) for _ in range(NBUF)),
        tuple(pltpu.VMEM((CH, H), jnp.float32) for _ in range(NBUF)),
        pltpu.VMEM_SHARED((N, H), jnp.float32),
        tuple(pltpu.SemaphoreType.DMA for _ in range(NBUF)),
        tuple(pltpu.SemaphoreType.DMA for _ in range(NBUF)),
        tuple(pltpu.SemaphoreType.DMA for _ in range(NBUF)),
    ],
)


_RB = 1000


def _mm_body(x_ref, w_ref, dis_ref, y_ref):
    xw = jnp.dot(x_ref[...], w_ref[...], preferred_element_type=jnp.float32)
    y = dis_ref[...] * xw
    y_ref[0] = y[:, :H]
    y_ref[1] = y[:, H:]


def _mm_kernel(x, w, dis2):
    return pl.pallas_call(
        _mm_body,
        grid=(N // _RB,),
        in_specs=[
            pl.BlockSpec((_RB, D), lambda i: (i, 0)),
            pl.BlockSpec((D, D), lambda i: (0, 0)),
            pl.BlockSpec((_RB, 1), lambda i: (i, 0)),
        ],
        out_specs=pl.BlockSpec((NC, _RB, H), lambda i: (0, i, 0)),
        out_shape=jax.ShapeDtypeStruct((NC, N, H), jnp.float32),
    )(x, w, dis2)


def _mid_body(zp_ref, dis_ref, b_ref, w_ref, y2_ref):
    z = jnp.concatenate([zp_ref[0], zp_ref[1]], axis=1)
    h = jnp.maximum(dis_ref[...] * z + b_ref[...], 0.0)
    hw = jnp.dot(h, w_ref[...], preferred_element_type=jnp.float32)
    y2 = dis_ref[...] * hw
    y2_ref[0] = y2[:, :H]
    y2_ref[1] = y2[:, H:]


def _mid_kernel(zp, dis2, b1, w2):
    return pl.pallas_call(
        _mid_body,
        grid=(N // _RB,),
        in_specs=[
            pl.BlockSpec((NC, _RB, H), lambda i: (0, i, 0)),
            pl.BlockSpec((_RB, 1), lambda i: (i, 0)),
            pl.BlockSpec((1, D), lambda i: (0, 0)),
            pl.BlockSpec((D, D), lambda i: (0, 0)),
        ],
        out_specs=pl.BlockSpec((NC, _RB, H), lambda i: (0, i, 0)),
        out_shape=jax.ShapeDtypeStruct((NC, N, H), jnp.float32),
    )(zp, dis2, b1, w2)


def _out_body(zp_ref, dis_ref, b_ref, o_ref):
    z = jnp.concatenate([zp_ref[0], zp_ref[1]], axis=1)
    o_ref[...] = dis_ref[...] * z + b_ref[...]


def _out_kernel(zp, dis2, b2):
    return pl.pallas_call(
        _out_body,
        grid=(N // _RB,),
        in_specs=[
            pl.BlockSpec((NC, _RB, H), lambda i: (0, i, 0)),
            pl.BlockSpec((_RB, 1), lambda i: (i, 0)),
            pl.BlockSpec((1, D), lambda i: (0, 0)),
        ],
        out_specs=pl.BlockSpec((_RB, D), lambda i: (i, 0)),
        out_shape=jax.ShapeDtypeStruct((N, D), jnp.float32),
    )(zp, dis2, b2)


def kernel(x, edge_index, W1, b1, W2, b2):
    ei = edge_index.astype(jnp.int32)
    row2d = ei[0].reshape(E // CH, CH)
    col2d = ei[1].reshape(E // CH, CH)

    dis2 = _deg_kernel(ei[1]).reshape(N, 1)

    y1 = _mm_kernel(x, W1, dis2)
    zp1 = _edge_kernel(y1, row2d, col2d)
    y2 = _mid_kernel(zp1, dis2, b1.reshape(1, D), W2)
    zp2 = _edge_kernel(y2, row2d, col2d)
    return _out_kernel(zp2, dis2, b2.reshape(1, D))


# speedup vs baseline: 34.4763x; 1.0541x over previous
import jax
import jax.numpy as jnp
from jax import lax
from jax.experimental import pallas as pl
from jax.experimental.pallas import tpu as pltpu
from jax.experimental.pallas import tpu_sc as plsc

N = 10000
E = 320000
D = 128
H = D // 2

NC, NS, L = 2, 16, 16
NT = NC * NS
EPT = E // NT
EPS = E // NS
CH = 125
NCHUNK = EPS // CH
SUB = 624

_MESH = plsc.VectorSubcoreMesh(
    core_axis_name="c", subcore_axis_name="s", num_cores=NC, num_subcores=NS)


def _rsqrt16(d):
    i = plsc.bitcast(d, jnp.int32)
    i = 0x5F3759DF - lax.shift_right_arithmetic(i, 1)
    y = plsc.bitcast(i, jnp.float32)
    for _ in range(3):
        y = y * (1.5 - 0.5 * d * y * y)
    return y


def _deg_body(col_hbm, dis_hbm, col_v, hist, tmp, acc, hists_sh):
    c = lax.axis_index("c")
    s = lax.axis_index("s")
    pltpu.sync_copy(col_hbm.at[pl.ds(s * EPS, EPS)], col_v)

    def zero(i, carry):
        hist[pl.ds(i * L, L)] = jnp.zeros((L,), jnp.float32)
        return carry

    lax.fori_loop(0, N // L, zero, 0)

    ones = jnp.ones((L,), jnp.float32)

    def bump(i, carry):
        idx = col_v[pl.ds(i * L, L)]
        plsc.addupdate_scatter(hist, [idx], ones)
        return carry

    lax.fori_loop(0, EPS // L, bump, 0)
    pltpu.sync_copy(hist, hists_sh.at[s])
    plsc.subcore_barrier()

    def reduce_rows(lo, npart, buf_lo):
        def zacc(i, carry):
            acc[pl.ds(buf_lo + i * L, L)] = jnp.zeros((L,), jnp.float32)
            return carry

        lax.fori_loop(0, npart // L, zacc, 0)
        for k in range(NS):
            pltpu.sync_copy(hists_sh.at[k, pl.ds(lo, npart)],
                            tmp.at[pl.ds(buf_lo, npart)])

            def addk(i, carry):
                sl = pl.ds(buf_lo + i * L, L)
                acc[sl] = acc[sl] + tmp[sl]
                return carry

            lax.fori_loop(0, npart // L, addk, 0)

        def finish(i, carry):
            sl = pl.ds(buf_lo + i * L, L)
            acc[sl] = _rsqrt16(acc[sl] + 1.0)
            return carry

        lax.fori_loop(0, npart // L, finish, 0)

    reduce_rows(s * SUB, SUB, 0)

    @pl.when(s == 0)
    def _():
        reduce_rows(NS * SUB, N - NS * SUB, SUB)

    @pl.when(c == 0)
    def _():
        pltpu.sync_copy(acc.at[pl.ds(0, SUB)], dis_hbm.at[pl.ds(s * SUB, SUB)])

        @pl.when(s == 0)
        def _():
            pltpu.sync_copy(acc.at[pl.ds(SUB, N - NS * SUB)],
                            dis_hbm.at[pl.ds(NS * SUB, N - NS * SUB)])


_deg_kernel = pl.kernel(
    _deg_body,
    out_type=jax.ShapeDtypeStruct((N,), jnp.float32),
    mesh=_MESH,
    compiler_params=pltpu.CompilerParams(needs_layout_passes=False,
                                         use_tc_tiling_on_sc=False),
    scratch_types=[
        pltpu.VMEM((EPS,), jnp.int32),
        pltpu.VMEM((N,), jnp.float32),
        pltpu.VMEM((SUB + 16,), jnp.float32),
        pltpu.VMEM((SUB + 16,), jnp.float32),
        pltpu.VMEM_SHARED((NS, N), jnp.float32),
    ],
)


NBUF = 8
PREF = 6


RCH = 104


def _edge_body(relu, y_hbm, row_hbm, col_hbm, dis_hbm, b_hbm, out_hbm,
               row_v, cbufs, gbufs, disb, bb, zsh, gsems, csems, ssems):
    c = lax.axis_index("c")
    s = lax.axis_index("s")
    yc = y_hbm.at[c]
    pltpu.sync_copy(row_hbm.at[pl.ds(s * NCHUNK, NCHUNK)], row_v)

    pltpu.sync_copy(yc.at[pl.ds(s * SUB, SUB)], zsh.at[pl.ds(s * SUB, SUB)])

    @pl.when(s == 0)
    def _():
        pltpu.sync_copy(yc.at[pl.ds(NS * SUB, N - NS * SUB)],
                        zsh.at[pl.ds(NS * SUB, N - NS * SUB)])

    plsc.subcore_barrier()

    def issue(j, b):
        pltpu.async_copy(yc.at[row_v.at[j]], gbufs[b], gsems[b])
        pltpu.async_copy(col_hbm.at[s * NCHUNK + j], cbufs[b], csems[b])

    for j in range(PREF):
        issue(j, j)

    def step(i, carry):
        j0 = NBUF * i
        for b in range(NBUF):
            j = j0 + b
            pltpu.make_async_copy(yc.at[row_v.at[j]], gbufs[b],
                                  gsems[b]).wait()
            pltpu.make_async_copy(col_hbm.at[s * NCHUNK + j], cbufs[b],
                                  csems[b]).wait()
            pltpu.async_copy(gbufs[b], zsh.at[cbufs[b]], ssems[b], add=True)
            b2 = (b + PREF) % NBUF

            @pl.when(j + PREF < NCHUNK)
            def _():
                @pl.when(j >= NBUF - PREF)
                def _():
                    pltpu.make_async_copy(
                        gbufs[b2], zsh.at[cbufs[b2]], ssems[b2]).wait()

                issue(j + PREF, b2)

        return carry

    lax.fori_loop(0, NCHUNK // NBUF, step, 0)
    for b in range(NBUF):
        pltpu.make_async_copy(gbufs[b], zsh.at[cbufs[b]], ssems[b]).wait()
    plsc.subcore_barrier()

    pltpu.sync_copy(dis_hbm.at[pl.ds(s * SUB, SUB)], disb.at[pl.ds(0, SUB)])
    pltpu.sync_copy(b_hbm.at[c], bb)
    bvs = [bb[pl.ds(k * L, L)] for k in range(H // L)]

    def emit_rows(lo, npart, buf_lo):
        e = gbufs[0]
        rn = min(RCH, npart)
        for q in range(npart // rn):
            r0 = lo + q * rn
            pltpu.sync_copy(zsh.at[pl.ds(r0, rn)], e.at[pl.ds(0, rn)])

            def rows(r, carry, _q=q):
                idx = jnp.full((L,), buf_lo + _q * rn, jnp.int32) + r
                dv = plsc.load_gather(disb, [idx])
                for k in range(H // L):
                    v = e[r, pl.ds(k * L, L)] * dv + bvs[k]
                    if relu:
                        v = jnp.maximum(v, 0.0)
                    e[r, pl.ds(k * L, L)] = v
                return carry

            lax.fori_loop(0, rn, rows, 0)
            pltpu.sync_copy(e.at[pl.ds(0, rn)],
                            out_hbm.at[pl.ds(r0, rn), pl.ds(c * H, H)])

    emit_rows(s * SUB, SUB, 0)

    @pl.when(s == 0)
    def _():
        pltpu.sync_copy(dis_hbm.at[pl.ds(NS * SUB, N - NS * SUB)],
                        disb.at[pl.ds(SUB, N - NS * SUB)])
        emit_rows(NS * SUB, N - NS * SUB, SUB)


def _make_edge(relu):
    import functools as _ft
    return pl.kernel(
        _ft.partial(_edge_body, relu),
        out_type=jax.ShapeDtypeStruct((N, D), jnp.float32),
        mesh=_MESH,
        compiler_params=pltpu.CompilerParams(use_tc_tiling_on_sc=False,
                                             needs_layout_passes=False),
        scratch_types=[
            pltpu.VMEM((NCHUNK, CH), jnp.int32),
            tuple(pltpu.VMEM((CH,), jnp.int32) for _ in range(NBUF)),
            tuple(pltpu.VMEM((CH, H), jnp.float32) for _ in range(NBUF)),
            pltpu.VMEM((SUB + 16,), jnp.float32),
            pltpu.VMEM((H,), jnp.float32),
            pltpu.VMEM_SHARED((N, H), jnp.float32),
            tuple(pltpu.SemaphoreType.DMA for _ in range(NBUF)),
            tuple(pltpu.SemaphoreType.DMA for _ in range(NBUF)),
            tuple(pltpu.SemaphoreType.DMA for _ in range(NBUF)),
        ],
    )


_edge_relu = _make_edge(True)
_edge_plain = _make_edge(False)


_RB = 1000


def _mm_body(x_ref, w_ref, dis_ref, y_ref):
    xw = jnp.dot(x_ref[...], w_ref[...], preferred_element_type=jnp.float32)
    y = dis_ref[...] * xw
    y_ref[0] = y[:, :H]
    y_ref[1] = y[:, H:]


def _mm_kernel(x, w, dis2):
    return pl.pallas_call(
        _mm_body,
        grid=(N // _RB,),
        in_specs=[
            pl.BlockSpec((_RB, D), lambda i: (i, 0)),
            pl.BlockSpec((D, D), lambda i: (0, 0)),
            pl.BlockSpec((_RB, 1), lambda i: (i, 0)),
        ],
        out_specs=pl.BlockSpec((NC, _RB, H), lambda i: (0, i, 0)),
        out_shape=jax.ShapeDtypeStruct((NC, N, H), jnp.float32),
    )(x, w, dis2)


def kernel(x, edge_index, W1, b1, W2, b2):
    ei = edge_index.astype(jnp.int32)
    row2d = ei[0].reshape(E // CH, CH)
    col2d = ei[1].reshape(E // CH, CH)

    dis = _deg_kernel(ei[1])
    dis2 = dis.reshape(N, 1)

    y1 = _mm_kernel(x, W1, dis2)
    h = _edge_relu(y1, row2d, col2d, dis, b1.reshape(NC, H))
    y2 = _mm_kernel(h, W2, dis2)
    return _edge_plain(y2, row2d, col2d, dis, b2.reshape(NC, H))


# speedup vs baseline: 37.2641x; 1.0809x over previous
import jax
import jax.numpy as jnp
from jax import lax
from jax.experimental import pallas as pl
from jax.experimental.pallas import tpu as pltpu
from jax.experimental.pallas import tpu_sc as plsc

N = 10000
E = 320000
D = 128
H = D // 2

NC, NS, L = 2, 16, 16
NT = NC * NS
EPT = E // NT
EPS = E // NS
CH = 125
NCHUNK = EPS // CH
SUB = 624

_MESH = plsc.VectorSubcoreMesh(
    core_axis_name="c", subcore_axis_name="s", num_cores=NC, num_subcores=NS)


def _rsqrt16(d):
    i = plsc.bitcast(d, jnp.int32)
    i = 0x5F3759DF - lax.shift_right_arithmetic(i, 1)
    y = plsc.bitcast(i, jnp.float32)
    for _ in range(3):
        y = y * (1.5 - 0.5 * d * y * y)
    return y


def _deg_body(col_hbm, dis_hbm, col_v, hist, tmp, acc, hists_sh):
    c = lax.axis_index("c")
    s = lax.axis_index("s")
    pltpu.sync_copy(col_hbm.at[pl.ds(s * EPS, EPS)], col_v)

    @plsc.parallel_loop(0, N // L, unroll=8)
    def _(i):
        hist[pl.ds(i * L, L)] = jnp.zeros((L,), jnp.float32)

    ones = jnp.ones((L,), jnp.float32)

    @plsc.parallel_loop(0, EPS // L, unroll=8)
    def _(i):
        idx = col_v[pl.ds(i * L, L)]
        plsc.addupdate_scatter(hist, [idx], ones)
    pltpu.sync_copy(hist, hists_sh.at[s])
    plsc.subcore_barrier()

    def reduce_rows(lo, npart, buf_lo):
        @plsc.parallel_loop(0, npart // L, unroll=4)
        def _(i):
            acc[pl.ds(buf_lo + i * L, L)] = jnp.zeros((L,), jnp.float32)

        for k in range(NS):
            pltpu.sync_copy(hists_sh.at[k, pl.ds(lo, npart)],
                            tmp.at[pl.ds(buf_lo, npart)])

            @plsc.parallel_loop(0, npart // L, unroll=4)
            def _(i):
                sl = pl.ds(buf_lo + i * L, L)
                acc[sl] = acc[sl] + tmp[sl]

        @plsc.parallel_loop(0, npart // L, unroll=4)
        def _(i):
            sl = pl.ds(buf_lo + i * L, L)
            acc[sl] = _rsqrt16(acc[sl] + 1.0)

    reduce_rows(s * SUB, SUB, 0)

    @pl.when(s == 0)
    def _():
        reduce_rows(NS * SUB, N - NS * SUB, SUB)

    @pl.when(c == 0)
    def _():
        pltpu.sync_copy(acc.at[pl.ds(0, SUB)], dis_hbm.at[pl.ds(s * SUB, SUB)])

        @pl.when(s == 0)
        def _():
            pltpu.sync_copy(acc.at[pl.ds(SUB, N - NS * SUB)],
                            dis_hbm.at[pl.ds(NS * SUB, N - NS * SUB)])


_deg_kernel = pl.kernel(
    _deg_body,
    out_type=jax.ShapeDtypeStruct((N,), jnp.float32),
    mesh=_MESH,
    compiler_params=pltpu.CompilerParams(needs_layout_passes=False,
                                         use_tc_tiling_on_sc=False),
    scratch_types=[
        pltpu.VMEM((EPS,), jnp.int32),
        pltpu.VMEM((N,), jnp.float32),
        pltpu.VMEM((SUB + 16,), jnp.float32),
        pltpu.VMEM((SUB + 16,), jnp.float32),
        pltpu.VMEM_SHARED((NS, N), jnp.float32),
    ],
)


NBUF = 8
PREF = 6


RCH = 104


def _edge_body(relu, y_hbm, row_hbm, col_hbm, dis_hbm, b_hbm, out_hbm,
               row_v, cbufs, gbufs, disb, bb, zsh, gsems, csems, ssems):
    c = lax.axis_index("c")
    s = lax.axis_index("s")
    yc = y_hbm.at[c]
    pltpu.sync_copy(row_hbm.at[pl.ds(s * NCHUNK, NCHUNK)], row_v)

    pltpu.sync_copy(yc.at[pl.ds(s * SUB, SUB)], zsh.at[pl.ds(s * SUB, SUB)])

    @pl.when(s == 0)
    def _():
        pltpu.sync_copy(yc.at[pl.ds(NS * SUB, N - NS * SUB)],
                        zsh.at[pl.ds(NS * SUB, N - NS * SUB)])

    plsc.subcore_barrier()

    def issue(j, b):
        pltpu.async_copy(yc.at[row_v.at[j]], gbufs[b], gsems[b])
        pltpu.async_copy(col_hbm.at[s * NCHUNK + j], cbufs[b], csems[b])

    for j in range(PREF):
        issue(j, j)

    def step(i, carry):
        j0 = NBUF * i
        for b in range(NBUF):
            j = j0 + b
            pltpu.make_async_copy(yc.at[row_v.at[j]], gbufs[b],
                                  gsems[b]).wait()
            pltpu.make_async_copy(col_hbm.at[s * NCHUNK + j], cbufs[b],
                                  csems[b]).wait()
            pltpu.async_copy(gbufs[b], zsh.at[cbufs[b]], ssems[b], add=True)
            b2 = (b + PREF) % NBUF

            @pl.when(j + PREF < NCHUNK)
            def _():
                @pl.when(j >= NBUF - PREF)
                def _():
                    pltpu.make_async_copy(
                        gbufs[b2], zsh.at[cbufs[b2]], ssems[b2]).wait()

                issue(j + PREF, b2)

        return carry

    lax.fori_loop(0, NCHUNK // NBUF, step, 0)
    for b in range(NBUF):
        pltpu.make_async_copy(gbufs[b], zsh.at[cbufs[b]], ssems[b]).wait()
    plsc.subcore_barrier()

    pltpu.sync_copy(dis_hbm.at[pl.ds(s * SUB, SUB)], disb.at[pl.ds(0, SUB)])
    pltpu.sync_copy(b_hbm.at[c], bb)
    bvs = [bb[pl.ds(k * L, L)] for k in range(H // L)]

    def emit_rows(lo, npart, buf_lo):
        e = gbufs[0]
        rn = min(RCH, npart)
        for q in range(npart // rn):
            r0 = lo + q * rn
            pltpu.sync_copy(zsh.at[pl.ds(r0, rn)], e.at[pl.ds(0, rn)])

            @plsc.parallel_loop(0, rn, unroll=4)
            def _(r, _q=q):
                idx = jnp.full((L,), buf_lo + _q * rn, jnp.int32) + r
                dv = plsc.load_gather(disb, [idx])
                for k in range(H // L):
                    v = e[r, pl.ds(k * L, L)] * dv + bvs[k]
                    if relu:
                        v = jnp.maximum(v, 0.0)
                    e[r, pl.ds(k * L, L)] = v
            pltpu.sync_copy(e.at[pl.ds(0, rn)],
                            out_hbm.at[pl.ds(r0, rn), pl.ds(c * H, H)])

    emit_rows(s * SUB, SUB, 0)

    @pl.when(s == 0)
    def _():
        pltpu.sync_copy(dis_hbm.at[pl.ds(NS * SUB, N - NS * SUB)],
                        disb.at[pl.ds(SUB, N - NS * SUB)])
        emit_rows(NS * SUB, N - NS * SUB, SUB)


def _make_edge(relu):
    import functools as _ft
    return pl.kernel(
        _ft.partial(_edge_body, relu),
        out_type=jax.ShapeDtypeStruct((N, D), jnp.float32),
        mesh=_MESH,
        compiler_params=pltpu.CompilerParams(use_tc_tiling_on_sc=False,
                                             needs_layout_passes=False),
        scratch_types=[
            pltpu.VMEM((NCHUNK, CH), jnp.int32),
            tuple(pltpu.VMEM((CH,), jnp.int32) for _ in range(NBUF)),
            tuple(pltpu.VMEM((CH, H), jnp.float32) for _ in range(NBUF)),
            pltpu.VMEM((SUB + 16,), jnp.float32),
            pltpu.VMEM((H,), jnp.float32),
            pltpu.VMEM_SHARED((N, H), jnp.float32),
            tuple(pltpu.SemaphoreType.DMA for _ in range(NBUF)),
            tuple(pltpu.SemaphoreType.DMA for _ in range(NBUF)),
            tuple(pltpu.SemaphoreType.DMA for _ in range(NBUF)),
        ],
    )


_edge_relu = _make_edge(True)
_edge_plain = _make_edge(False)


_RB = 1000


def _mm_body(x_ref, w_ref, dis_ref, y_ref):
    xw = jnp.dot(x_ref[...], w_ref[...], preferred_element_type=jnp.float32)
    y = dis_ref[...] * xw
    y_ref[0] = y[:, :H]
    y_ref[1] = y[:, H:]


def _mm_kernel(x, w, dis2):
    return pl.pallas_call(
        _mm_body,
        grid=(N // _RB,),
        in_specs=[
            pl.BlockSpec((_RB, D), lambda i: (i, 0)),
            pl.BlockSpec((D, D), lambda i: (0, 0)),
            pl.BlockSpec((_RB, 1), lambda i: (i, 0)),
        ],
        out_specs=pl.BlockSpec((NC, _RB, H), lambda i: (0, i, 0)),
        out_shape=jax.ShapeDtypeStruct((NC, N, H), jnp.float32),
    )(x, w, dis2)


def kernel(x, edge_index, W1, b1, W2, b2):
    ei = edge_index.astype(jnp.int32)
    row2d = ei[0].reshape(E // CH, CH)
    col2d = ei[1].reshape(E // CH, CH)

    dis = _deg_kernel(ei[1])
    dis2 = dis.reshape(N, 1)

    y1 = _mm_kernel(x, W1, dis2)
    h = _edge_relu(y1, row2d, col2d, dis, b1.reshape(NC, H))
    y2 = _mm_kernel(h, W2, dis2)
    return _edge_plain(y2, row2d, col2d, dis, b2.reshape(NC, H))
